# Initial kernel scaffold; baseline (speedup 1.0000x reference)
import jax
import jax.numpy as jnp
from jax.experimental import pallas as pl


def kernel(x, table):
    raise NotImplementedError("write your pallas kernel here")


# speedup vs baseline: 2.6826x; 2.6826x over previous
import jax
import jax.numpy as jnp
from jax.experimental import pallas as pl


def _body(t_ref, o_ref):
    o_ref[...] = jnp.broadcast_to(t_ref[...][None, :, :], o_ref.shape)


def kernel(x, table):
    batch, seq_len, d = x.shape
    s_blk = 512
    grid = (seq_len // s_blk,)
    return pl.pallas_call(
        _body,
        grid=grid,
        in_specs=[pl.BlockSpec((s_blk, d), lambda i: (i, 0))],
        out_specs=pl.BlockSpec((batch, s_blk, d), lambda i: (0, i, 0)),
        out_shape=jax.ShapeDtypeStruct((batch, seq_len, d), x.dtype),
    )(table)
